# baseline (speedup 1.0000x reference)
import jax
import jax.numpy as jnp
from jax.experimental import pallas as pl


BLOCK_F = 4


def _add_kernel(x_ref, t_ref, o_ref):
    i = pl.program_id(0)
    t_blk = t_ref[pl.ds(i * BLOCK_F, BLOCK_F), :]
    o_ref[...] = x_ref[...] + t_blk[:, :, None]


def kernel(inputs, table):
    b, f, d = inputs.shape
    x3 = jnp.transpose(inputs, (1, 2, 0))

    out3 = pl.pallas_call(
        _add_kernel,
        grid=(f // BLOCK_F,),
        in_specs=[
            pl.BlockSpec((BLOCK_F, d, b), lambda i: (i, 0, 0)),
            pl.BlockSpec((f, d), lambda i: (0, 0)),
        ],
        out_specs=pl.BlockSpec((BLOCK_F, d, b), lambda i: (i, 0, 0)),
        out_shape=jax.ShapeDtypeStruct((f, d, b), inputs.dtype),
    )(x3, table)
    return jnp.transpose(out3, (2, 0, 1))


# speedup vs baseline: 1.0030x; 1.0030x over previous
import jax
import jax.numpy as jnp
from jax.experimental import pallas as pl


BLOCK_F = 5


def _add_kernel(x_ref, t_ref, o_ref):
    i = pl.program_id(0)
    t_blk = t_ref[pl.ds(i * BLOCK_F, BLOCK_F), :]
    o_ref[...] = x_ref[...] + t_blk[:, :, None]


def kernel(inputs, table):
    b, f, d = inputs.shape
    x3 = jnp.transpose(inputs, (1, 2, 0))

    out3 = pl.pallas_call(
        _add_kernel,
        grid=(f // BLOCK_F,),
        in_specs=[
            pl.BlockSpec((BLOCK_F, d, b), lambda i: (i, 0, 0)),
            pl.BlockSpec((f, d), lambda i: (0, 0)),
        ],
        out_specs=pl.BlockSpec((BLOCK_F, d, b), lambda i: (i, 0, 0)),
        out_shape=jax.ShapeDtypeStruct((f, d, b), inputs.dtype),
    )(x3, table)
    return jnp.transpose(out3, (2, 0, 1))


# speedup vs baseline: 1.0031x; 1.0001x over previous
import jax
import jax.numpy as jnp
from jax.experimental import pallas as pl
from jax.experimental.pallas import tpu as pltpu


BLOCK_F = 5


def _add_kernel(x_ref, t_ref, o_ref):
    i = pl.program_id(0)
    t_blk = t_ref[pl.ds(i * BLOCK_F, BLOCK_F), :]
    o_ref[...] = x_ref[...] + t_blk[:, :, None]


def kernel(inputs, table):
    b, f, d = inputs.shape
    x3 = jnp.transpose(inputs, (1, 2, 0))

    out3 = pl.pallas_call(
        _add_kernel,
        grid=(f // BLOCK_F,),
        in_specs=[
            pl.BlockSpec((BLOCK_F, d, b), lambda i: (i, 0, 0)),
            pl.BlockSpec((f, d), lambda i: (0, 0)),
        ],
        out_specs=pl.BlockSpec((BLOCK_F, d, b), lambda i: (i, 0, 0)),
        out_shape=jax.ShapeDtypeStruct((f, d, b), inputs.dtype),
        compiler_params=pltpu.CompilerParams(
            dimension_semantics=("parallel",),
        ),
    )(x3, table)
    return jnp.transpose(out3, (2, 0, 1))
